# baseline (speedup 1.0000x reference)
import functools

import jax
import jax.numpy as jnp
from jax import lax
from jax.experimental import pallas as pl
from jax.experimental.pallas import tpu as pltpu
from jax.experimental.pallas import tpu_sc as plsc


def _make_gather(N, V, D, num_cores, num_subcores):
    NW = num_cores * num_subcores
    per_w = N // NW
    IW = 128
    CH = 1024
    n_sub = CH // IW
    n_ch = per_w // CH
    mesh = plsc.VectorSubcoreMesh(core_axis_name="c", subcore_axis_name="s")

    @functools.partial(
        pl.kernel,
        out_type=jax.ShapeDtypeStruct((N, D), jnp.float32),
        mesh=mesh,
        scratch_types=[
            pltpu.VMEM((n_sub, IW), jnp.int32),
            pltpu.VMEM((CH, D), jnp.float32),
            pltpu.SemaphoreType.DMA,
        ],
        compiler_params=pltpu.CompilerParams(use_tc_tiling_on_sc=False),
    )
    def gather_kernel(idx_hbm, tbl_hbm, out_hbm, idx_v, rows_v, sem):
        wid = lax.axis_index("s") * num_cores + lax.axis_index("c")
        base = wid * per_w

        def body(i, carry):
            off = base + i * CH
            row_off = pl.multiple_of(off // IW, 8)
            pltpu.sync_copy(
                idx_hbm.at[pl.ds(row_off, n_sub)],
                idx_v,
            )
            descs = [
                pltpu.async_copy(
                    tbl_hbm.at[idx_v.at[j]],
                    rows_v.at[pl.ds(j * IW, IW)],
                    sem,
                )
                for j in range(n_sub)
            ]
            for d in descs:
                d.wait()
            pltpu.sync_copy(rows_v, out_hbm.at[pl.ds(off, CH)])
            return carry

        lax.fori_loop(0, n_ch, body, 0)

    return gather_kernel


def kernel(x, table):
    B, F = x.shape
    V, D = table.shape
    N = B * F
    idx2 = x.reshape(N // 128, 128).astype(jnp.int32)
    gather = _make_gather(N, V, D, 2, 16)
    out = gather(idx2, table)
    return out.reshape(B, F * D)


# speedup vs baseline: 1.0105x; 1.0105x over previous
import functools

import jax
import jax.numpy as jnp
from jax import lax
from jax.experimental import pallas as pl
from jax.experimental.pallas import tpu as pltpu
from jax.experimental.pallas import tpu_sc as plsc

_NBUF = 2


def _make_gather(N, V, D, num_cores, num_subcores):
    NW = num_cores * num_subcores
    per_w = N // NW
    IW = 128
    CH = 512
    n_sub = CH // IW
    n_ch = per_w // CH
    assert n_ch % _NBUF == 0
    mesh = plsc.VectorSubcoreMesh(core_axis_name="c", subcore_axis_name="s")

    @functools.partial(
        pl.kernel,
        out_type=jax.ShapeDtypeStruct((N, D), jnp.float32),
        mesh=mesh,
        scratch_types=[
            pltpu.VMEM((_NBUF, CH), jnp.int32),
            pltpu.VMEM((_NBUF, CH, D), jnp.float32),
            [pltpu.SemaphoreType.DMA] * _NBUF,
            [pltpu.SemaphoreType.DMA] * _NBUF,
        ],
        compiler_params=pltpu.CompilerParams(use_tc_tiling_on_sc=False),
    )
    def gather_kernel(idx_hbm, tbl_hbm, out_hbm, idx_v, rows_v, gsems, ssems):
        wid = lax.axis_index("s") * num_cores + lax.axis_index("c")
        base = wid * per_w

        def load_idx(i, b):
            pltpu.sync_copy(idx_hbm.at[pl.ds(base + i * CH, CH)], idx_v.at[b])

        def fire_gather(b):
            for j in range(n_sub):
                pltpu.async_copy(
                    tbl_hbm.at[idx_v.at[b].at[pl.ds(j * IW, IW)]],
                    rows_v.at[b].at[pl.ds(j * IW, IW)],
                    gsems[b],
                )

        def wait_gather(b):
            for j in range(n_sub):
                pltpu.make_async_copy(
                    tbl_hbm.at[idx_v.at[b].at[pl.ds(j * IW, IW)]],
                    rows_v.at[b].at[pl.ds(j * IW, IW)],
                    gsems[b],
                ).wait()

        def fire_store(i, b):
            pltpu.async_copy(
                rows_v.at[b], out_hbm.at[pl.ds(base + i * CH, CH)], ssems[b]
            )

        def wait_store(i, b):
            pltpu.make_async_copy(
                rows_v.at[b], out_hbm.at[pl.ds(base + i * CH, CH)], ssems[b]
            ).wait()

        for b in range(_NBUF):
            load_idx(b, b)
            fire_gather(b)

        def body(it, carry):
            i0 = it * _NBUF
            for b in range(_NBUF):
                i = i0 + b
                wait_gather(b)
                fire_store(i, b)
                @pl.when(i + _NBUF < n_ch)
                def _():
                    load_idx(i + _NBUF, b)
                    wait_store(i, b)
                    fire_gather(b)
            return carry

        lax.fori_loop(0, n_ch // _NBUF, body, 0)

        for b in range(_NBUF):
            wait_store(n_ch - _NBUF + b, b)

    return gather_kernel


def kernel(x, table):
    B, F = x.shape
    V, D = table.shape
    N = B * F
    flat_idx = x.reshape(N).astype(jnp.int32)
    gather = _make_gather(N, V, D, 2, 16)
    out = gather(flat_idx, table)
    return out.reshape(B, F * D)


# speedup vs baseline: 1.0118x; 1.0013x over previous
import functools

import jax
import jax.numpy as jnp
from jax import lax
from jax.experimental import pallas as pl
from jax.experimental.pallas import tpu as pltpu
from jax.experimental.pallas import tpu_sc as plsc

_NBUF = 4


def _make_gather(N, V, D, num_cores, num_subcores):
    NW = num_cores * num_subcores
    per_w = N // NW
    IW = 104
    CH = 416
    n_sub = CH // IW
    n_ch = per_w // CH
    assert n_ch % _NBUF == 0
    mesh = plsc.VectorSubcoreMesh(core_axis_name="c", subcore_axis_name="s")

    @functools.partial(
        pl.kernel,
        out_type=jax.ShapeDtypeStruct((N, D), jnp.float32),
        mesh=mesh,
        scratch_types=[
            pltpu.VMEM((per_w,), jnp.int32),
            pltpu.VMEM((_NBUF, CH, D), jnp.float32),
            [pltpu.SemaphoreType.DMA] * _NBUF,
            [pltpu.SemaphoreType.DMA] * _NBUF,
        ],
        compiler_params=pltpu.CompilerParams(use_tc_tiling_on_sc=False),
    )
    def gather_kernel(idx_hbm, tbl_hbm, out_hbm, idx_v, rows_v, gsems, ssems):
        wid = lax.axis_index("s") * num_cores + lax.axis_index("c")
        base = wid * per_w

        pltpu.sync_copy(idx_hbm.at[pl.ds(base, per_w)], idx_v)

        def fire_gather(i, b):
            for j in range(n_sub):
                pltpu.async_copy(
                    tbl_hbm.at[idx_v.at[pl.ds(i * CH + j * IW, IW)]],
                    rows_v.at[b].at[pl.ds(j * IW, IW)],
                    gsems[b],
                )

        def wait_gather(i, b):
            for j in range(n_sub):
                pltpu.make_async_copy(
                    tbl_hbm.at[idx_v.at[pl.ds(i * CH + j * IW, IW)]],
                    rows_v.at[b].at[pl.ds(j * IW, IW)],
                    gsems[b],
                ).wait()

        def fire_store(i, b):
            pltpu.async_copy(
                rows_v.at[b], out_hbm.at[pl.ds(base + i * CH, CH)], ssems[b]
            )

        def wait_store(i, b):
            pltpu.make_async_copy(
                rows_v.at[b], out_hbm.at[pl.ds(base + i * CH, CH)], ssems[b]
            ).wait()

        for b in range(_NBUF):
            fire_gather(b, b)

        def body(it, carry):
            i0 = it * _NBUF
            for b in range(_NBUF):
                i = i0 + b
                wait_gather(i, b)
                fire_store(i, b)
                @pl.when(i + _NBUF < n_ch)
                def _():
                    wait_store(i, b)
                    fire_gather(i + _NBUF, b)
            return carry

        lax.fori_loop(0, n_ch // _NBUF, body, 0)

        for b in range(_NBUF):
            wait_store(n_ch - _NBUF + b, b)

    return gather_kernel


def kernel(x, table):
    B, F = x.shape
    V, D = table.shape
    N = B * F
    flat_idx = x.reshape(N).astype(jnp.int32)
    gather = _make_gather(N, V, D, 2, 16)
    out = gather(flat_idx, table)
    return out.reshape(B, F * D)
